# Initial kernel scaffold; baseline (speedup 1.0000x reference)
import jax
import jax.numpy as jnp
from jax.experimental import pallas as pl


def kernel(input_ids, scores, allowed_ids):
    raise NotImplementedError("write your pallas kernel here")


# speedup vs baseline: 4.8583x; 4.8583x over previous
import jax
import jax.numpy as jnp
from jax.experimental import pallas as pl
from jax.experimental.pallas import tpu as pltpu

_LANE = 128
_BLOCK = 8192


def kernel(input_ids, scores, allowed_ids):
    del input_ids
    batch, vocab = scores.shape
    nids = allowed_ids.shape[0]

    def gather_body(ids_ref, scores_ref, out_ref):
        i = pl.program_id(0)
        c = ids_ref[i] % _LANE
        colmask = jax.lax.broadcasted_iota(jnp.int32, (batch, _LANE), 1) == c
        col = jnp.sum(jnp.where(colmask, scores_ref[...], 0.0), axis=1,
                      keepdims=True)

        @pl.when(i == 0)
        def _init():
            out_ref[...] = jnp.zeros_like(out_ref)

        slot = jax.lax.broadcasted_iota(jnp.int32, (batch, nids), 1) == i
        out_ref[...] = jnp.where(slot, col, out_ref[...])

    gathered = pl.pallas_call(
        gather_body,
        grid_spec=pltpu.PrefetchScalarGridSpec(
            num_scalar_prefetch=1,
            grid=(nids,),
            in_specs=[
                pl.BlockSpec((batch, _LANE), lambda i, ids: (0, ids[i] // _LANE)),
            ],
            out_specs=pl.BlockSpec((batch, nids), lambda i, ids: (0, 0)),
        ),
        out_shape=jax.ShapeDtypeStruct((batch, nids), scores.dtype),
    )(allowed_ids, scores)

    num_blocks = pl.cdiv(vocab, _BLOCK)

    def fill_body(ids_ref, gath_ref, out_ref):
        i = pl.program_id(0)
        base = i * _BLOCK
        out_ref[...] = jnp.full((batch, _BLOCK), -jnp.inf, out_ref.dtype)
        coliota = jax.lax.broadcasted_iota(jnp.int32, (batch, _BLOCK), 1)
        for j in range(nids):
            pos = ids_ref[j] - base

            @pl.when((pos >= 0) & (pos < _BLOCK))
            def _scatter(j=j, pos=pos):
                val = gath_ref[:, j:j + 1]
                out_ref[...] = jnp.where(coliota == pos, val, out_ref[...])

    out = pl.pallas_call(
        fill_body,
        grid_spec=pltpu.PrefetchScalarGridSpec(
            num_scalar_prefetch=1,
            grid=(num_blocks,),
            in_specs=[
                pl.BlockSpec((batch, nids), lambda i, ids: (0, 0)),
            ],
            out_specs=pl.BlockSpec((batch, _BLOCK), lambda i, ids: (0, i)),
        ),
        out_shape=jax.ShapeDtypeStruct((batch, vocab), scores.dtype),
    )(allowed_ids, gathered)
    return out


# speedup vs baseline: 6.0367x; 1.2426x over previous
import jax
import jax.numpy as jnp
from jax.experimental import pallas as pl
from jax.experimental.pallas import tpu as pltpu

_LANE = 128
_BLOCK = 32768


def kernel(input_ids, scores, allowed_ids):
    del input_ids
    batch, vocab = scores.shape
    nids = allowed_ids.shape[0]

    def gather_body(ids_ref, scores_ref, out_ref):
        i = pl.program_id(0)
        c = ids_ref[i] % _LANE
        colmask = jax.lax.broadcasted_iota(jnp.int32, (batch, _LANE), 1) == c
        col = jnp.sum(jnp.where(colmask, scores_ref[...], 0.0), axis=1,
                      keepdims=True)

        @pl.when(i == 0)
        def _init():
            out_ref[...] = jnp.zeros_like(out_ref)

        slot = jax.lax.broadcasted_iota(jnp.int32, (batch, nids), 1) == i
        out_ref[...] = jnp.where(slot, col, out_ref[...])

    gathered = pl.pallas_call(
        gather_body,
        grid_spec=pltpu.PrefetchScalarGridSpec(
            num_scalar_prefetch=1,
            grid=(nids,),
            in_specs=[
                pl.BlockSpec((batch, _LANE), lambda i, ids: (0, ids[i] // _LANE)),
            ],
            out_specs=pl.BlockSpec((batch, nids), lambda i, ids: (0, 0)),
        ),
        out_shape=jax.ShapeDtypeStruct((batch, nids), scores.dtype),
    )(allowed_ids, scores)

    num_blocks = pl.cdiv(vocab, _BLOCK)

    def fill_body(ids_ref, gath_ref, out_ref):
        i = pl.program_id(0)
        base = i * _BLOCK
        out_ref[...] = jnp.full((batch, _BLOCK), -jnp.inf, out_ref.dtype)
        coliota = jax.lax.broadcasted_iota(jnp.int32, (batch, _BLOCK), 1)
        for j in range(nids):
            pos = ids_ref[j] - base

            @pl.when((pos >= 0) & (pos < _BLOCK))
            def _scatter(j=j, pos=pos):
                val = gath_ref[:, j:j + 1]
                out_ref[...] = jnp.where(coliota == pos, val, out_ref[...])

    out = pl.pallas_call(
        fill_body,
        grid_spec=pltpu.PrefetchScalarGridSpec(
            num_scalar_prefetch=1,
            grid=(num_blocks,),
            in_specs=[
                pl.BlockSpec((batch, nids), lambda i, ids: (0, 0)),
            ],
            out_specs=pl.BlockSpec((batch, _BLOCK), lambda i, ids: (0, i)),
        ),
        out_shape=jax.ShapeDtypeStruct((batch, vocab), scores.dtype),
    )(allowed_ids, gathered)
    return out
